# baseline (speedup 1.0000x reference)
import jax
import jax.numpy as jnp
from jax import lax
from jax.experimental import pallas as pl
from jax.experimental.pallas import tpu as pltpu
from jax.experimental.pallas import tpu_sc as plsc

NUM_ENTITIES = 1000000
NUM_RELATIONS = 1000
EMBED_DIM = 64
BATCH = 16384

_NC = 2
_NS = 16
_NW = _NC * _NS
_BPW = BATCH // _NW
_SUB = 256
_NSUB = _BPW // _SUB
_CHUNK = 128
_NCHUNK = _SUB // _CHUNK
_L = 16
_GROUPS = _SUB // _L
_WIDE = 2 * EMBED_DIM


def _rsqrt_nr(s):
    s = jnp.maximum(s, jnp.float32(1e-30))
    i = plsc.bitcast(s, jnp.int32)
    i = jnp.int32(0x5F3759DF) - lax.shift_right_arithmetic(i, jnp.int32(1))
    y = plsc.bitcast(i, jnp.float32)
    half = jnp.float32(0.5)
    three_half = jnp.float32(1.5)
    for _ in range(3):
        y = y * (three_half - half * s * y * y)
    return y


def _transe_body(heads, relations, tails, ent2, rel2, out,
                 raw_h, raw_r, raw_t, idx_h, idx_r, idx_t,
                 h_rows, r_rows, t_rows, score_v, sem):
    wid = lax.axis_index("s") * _NC + lax.axis_index("c")
    base = wid * _BPW

    zero = jnp.zeros((_L,), jnp.float32)
    eps = jnp.float32(1e-12)
    lane = lax.iota(jnp.int32, _L)
    nchunks = EMBED_DIM // _L

    for s in range(_NSUB):
        sub_base = base + s * _SUB
        for j in range(_NCHUNK):
            off = sub_base + j * _CHUNK
            sl = pl.ds(off, _CHUNK)
            pltpu.sync_copy(heads.at[sl], raw_h.at[j])
            pltpu.sync_copy(relations.at[sl], raw_r.at[j])
            pltpu.sync_copy(tails.at[sl], raw_t.at[j])
        for j in range(_NCHUNK):
            for k in range(_CHUNK // _L):
                sl = pl.ds(k * _L, _L)
                idx_h[j, sl] = lax.shift_right_logical(raw_h[j, sl], 1)
                idx_r[j, sl] = lax.shift_right_logical(raw_r[j, sl], 1)
                idx_t[j, sl] = lax.shift_right_logical(raw_t[j, sl], 1)

        copies = []
        for j in range(_NCHUNK):
            dst = pl.ds(j * _CHUNK, _CHUNK)
            copies.append(pltpu.async_copy(ent2.at[idx_h.at[j]],
                                           h_rows.at[dst], sem))
            copies.append(pltpu.async_copy(rel2.at[idx_r.at[j]],
                                           r_rows.at[dst], sem))
            copies.append(pltpu.async_copy(ent2.at[idx_t.at[j]],
                                           t_rows.at[dst], sem))
        for c in copies:
            c.wait()

        def group(g, carry):
            cj = g // (_CHUNK // _L)
            co = (g % (_CHUNK // _L)) * _L
            par_h = lax.shift_left(raw_h[cj, pl.ds(co, _L)] & 1, 6)
            par_r = lax.shift_left(raw_r[cj, pl.ds(co, _L)] & 1, 6)
            par_t = lax.shift_left(raw_t[cj, pl.ds(co, _L)] & 1, 6)
            hh = rr = tt = hr = ht = rt = zero
            for j in range(_L):
                i = g * _L + j
                ph = par_h[j]
                pr = par_r[j]
                pt = par_t[j]
                hh_p = rr_p = tt_p = hr_p = ht_p = rt_p = zero
                for k in range(nchunks):
                    hv = h_rows[i, pl.ds(ph + k * _L, _L)]
                    rv = r_rows[i, pl.ds(pr + k * _L, _L)]
                    tv = t_rows[i, pl.ds(pt + k * _L, _L)]
                    hh_p = hh_p + hv * hv
                    rr_p = rr_p + rv * rv
                    tt_p = tt_p + tv * tv
                    hr_p = hr_p + hv * rv
                    ht_p = ht_p + hv * tv
                    rt_p = rt_p + rv * tv
                m = lane == j
                hh = jnp.where(m, jnp.sum(hh_p), hh)
                rr = jnp.where(m, jnp.sum(rr_p), rr)
                tt = jnp.where(m, jnp.sum(tt_p), tt)
                hr = jnp.where(m, jnp.sum(hr_p), hr)
                ht = jnp.where(m, jnp.sum(ht_p), ht)
                rt = jnp.where(m, jnp.sum(rt_p), rt)
            a = jnp.float32(1.0) / jnp.maximum(hh * _rsqrt_nr(hh), eps)
            b = jnp.float32(1.0) / jnp.maximum(rr * _rsqrt_nr(rr), eps)
            c = jnp.float32(1.0) / jnp.maximum(tt * _rsqrt_nr(tt), eps)
            s2 = (hh * a * a + rr * b * b + tt * c * c
                  + jnp.float32(2.0) * (a * b * hr - a * c * ht - b * c * rt))
            s2 = jnp.maximum(s2, jnp.float32(0.0))
            score_v[pl.ds(s * _SUB + g * _L, _L)] = s2 * _rsqrt_nr(s2)
            return carry

        lax.fori_loop(0, _GROUPS, group, 0)

    pltpu.sync_copy(score_v, out.at[pl.ds(base, _BPW)])


@jax.jit
def kernel(heads, relations, tails, entity_emb, relation_emb):
    ent2 = entity_emb.reshape(NUM_ENTITIES // 2, _WIDE)
    rel2 = relation_emb.reshape(NUM_RELATIONS // 2, _WIDE)
    mesh = plsc.VectorSubcoreMesh(core_axis_name="c", subcore_axis_name="s")
    f = pl.kernel(
        _transe_body,
        out_type=jax.ShapeDtypeStruct((BATCH,), jnp.float32),
        mesh=mesh,
        scratch_types=[
            pltpu.VMEM((_NCHUNK, _CHUNK), jnp.int32),
            pltpu.VMEM((_NCHUNK, _CHUNK), jnp.int32),
            pltpu.VMEM((_NCHUNK, _CHUNK), jnp.int32),
            pltpu.VMEM((_NCHUNK, _CHUNK), jnp.int32),
            pltpu.VMEM((_NCHUNK, _CHUNK), jnp.int32),
            pltpu.VMEM((_NCHUNK, _CHUNK), jnp.int32),
            pltpu.VMEM((_SUB, _WIDE), jnp.float32),
            pltpu.VMEM((_SUB, _WIDE), jnp.float32),
            pltpu.VMEM((_SUB, _WIDE), jnp.float32),
            pltpu.VMEM((_BPW,), jnp.float32),
            pltpu.SemaphoreType.DMA,
        ],
        compiler_params=pltpu.CompilerParams(needs_layout_passes=False),
    )
    return f(heads, relations, tails, ent2, rel2)


# speedup vs baseline: 1.6878x; 1.6878x over previous
import jax
import jax.numpy as jnp
from jax import lax
from jax.experimental import pallas as pl
from jax.experimental.pallas import tpu as pltpu
from jax.experimental.pallas import tpu_sc as plsc

NUM_ENTITIES = 1000000
NUM_RELATIONS = 1000
EMBED_DIM = 64
BATCH = 16384

_NC = 2
_NS = 16
_NW = _NC * _NS
_BPW = BATCH // _NW
_SUB = 256
_NSUB = _BPW // _SUB
_CHUNK = 128
_NCHUNK = _SUB // _CHUNK
_L = 16
_GROUPS = _SUB // _L


def _rsqrt_nr(s):
    s = jnp.maximum(s, jnp.float32(1e-30))
    i = plsc.bitcast(s, jnp.int32)
    i = jnp.int32(0x5F3759DF) - lax.shift_right_arithmetic(i, jnp.int32(1))
    y = plsc.bitcast(i, jnp.float32)
    half = jnp.float32(0.5)
    three_half = jnp.float32(1.5)
    for _ in range(3):
        y = y * (three_half - half * s * y * y)
    return y


def _transe_body(heads, relations, tails, entity_emb, relation_emb, out,
                 raw_h, raw_r, raw_t, h_rows, r_rows, t_rows, score_v, sem):
    wid = lax.axis_index("s") * _NC + lax.axis_index("c")
    base = wid * _BPW

    zero = jnp.zeros((_L,), jnp.float32)
    eps = jnp.float32(1e-12)
    lane = lax.iota(jnp.int32, _L)
    nchunks = EMBED_DIM // _L

    for s in range(_NSUB):
        sub_base = base + s * _SUB
        for j in range(_NCHUNK):
            sl = pl.ds(sub_base + j * _CHUNK, _CHUNK)
            pltpu.sync_copy(heads.at[sl], raw_h.at[j])
            pltpu.sync_copy(relations.at[sl], raw_r.at[j])
            pltpu.sync_copy(tails.at[sl], raw_t.at[j])

        def fire(g, carry):
            cj = g // (_CHUNK // _L)
            co = (g % (_CHUNK // _L)) * _L
            ih = raw_h[cj, pl.ds(co, _L)]
            ir = raw_r[cj, pl.ds(co, _L)]
            it = raw_t[cj, pl.ds(co, _L)]
            for j in range(_L):
                i = g * _L + j
                pltpu.make_async_copy(
                    entity_emb.at[pl.ds(ih[j], 1)],
                    h_rows.at[pl.ds(i, 1)], sem).start()
                pltpu.make_async_copy(
                    relation_emb.at[pl.ds(ir[j], 1)],
                    r_rows.at[pl.ds(i, 1)], sem).start()
                pltpu.make_async_copy(
                    entity_emb.at[pl.ds(it[j], 1)],
                    t_rows.at[pl.ds(i, 1)], sem).start()
            return carry

        lax.fori_loop(0, _GROUPS, fire, 0)
        pltpu.make_async_copy(
            entity_emb.at[pl.ds(0, _SUB)], h_rows, sem).wait()
        pltpu.make_async_copy(
            entity_emb.at[pl.ds(0, _SUB)], r_rows, sem).wait()
        pltpu.make_async_copy(
            entity_emb.at[pl.ds(0, _SUB)], t_rows, sem).wait()

        def group(g, carry):
            hh = rr = tt = hr = ht = rt = zero
            for j in range(_L):
                i = g * _L + j
                hh_p = rr_p = tt_p = hr_p = ht_p = rt_p = zero
                for k in range(nchunks):
                    sl = pl.ds(k * _L, _L)
                    hv = h_rows[i, sl]
                    rv = r_rows[i, sl]
                    tv = t_rows[i, sl]
                    hh_p = hh_p + hv * hv
                    rr_p = rr_p + rv * rv
                    tt_p = tt_p + tv * tv
                    hr_p = hr_p + hv * rv
                    ht_p = ht_p + hv * tv
                    rt_p = rt_p + rv * tv
                m = lane == j
                hh = jnp.where(m, jnp.sum(hh_p), hh)
                rr = jnp.where(m, jnp.sum(rr_p), rr)
                tt = jnp.where(m, jnp.sum(tt_p), tt)
                hr = jnp.where(m, jnp.sum(hr_p), hr)
                ht = jnp.where(m, jnp.sum(ht_p), ht)
                rt = jnp.where(m, jnp.sum(rt_p), rt)
            a = jnp.float32(1.0) / jnp.maximum(hh * _rsqrt_nr(hh), eps)
            b = jnp.float32(1.0) / jnp.maximum(rr * _rsqrt_nr(rr), eps)
            c = jnp.float32(1.0) / jnp.maximum(tt * _rsqrt_nr(tt), eps)
            s2 = (hh * a * a + rr * b * b + tt * c * c
                  + jnp.float32(2.0) * (a * b * hr - a * c * ht - b * c * rt))
            s2 = jnp.maximum(s2, jnp.float32(0.0))
            score_v[pl.ds(s * _SUB + g * _L, _L)] = s2 * _rsqrt_nr(s2)
            return carry

        lax.fori_loop(0, _GROUPS, group, 0)

    pltpu.sync_copy(score_v, out.at[pl.ds(base, _BPW)])


@jax.jit
def kernel(heads, relations, tails, entity_emb, relation_emb):
    mesh = plsc.VectorSubcoreMesh(core_axis_name="c", subcore_axis_name="s")
    f = pl.kernel(
        _transe_body,
        out_type=jax.ShapeDtypeStruct((BATCH,), jnp.float32),
        mesh=mesh,
        scratch_types=[
            pltpu.VMEM((_NCHUNK, _CHUNK), jnp.int32),
            pltpu.VMEM((_NCHUNK, _CHUNK), jnp.int32),
            pltpu.VMEM((_NCHUNK, _CHUNK), jnp.int32),
            pltpu.VMEM((_SUB, EMBED_DIM), jnp.float32),
            pltpu.VMEM((_SUB, EMBED_DIM), jnp.float32),
            pltpu.VMEM((_SUB, EMBED_DIM), jnp.float32),
            pltpu.VMEM((_BPW,), jnp.float32),
            pltpu.SemaphoreType.DMA,
        ],
        compiler_params=pltpu.CompilerParams(needs_layout_passes=False),
    )
    return f(heads, relations, tails, entity_emb, relation_emb)
